# baseline (speedup 1.0000x reference)
import functools

import jax
import jax.numpy as jnp
from jax import lax
from jax.experimental import pallas as pl
from jax.experimental.pallas import tpu as pltpu
from jax.experimental.pallas import tpu_sc as plsc

N_NODES = 10000
E_EDGES = 320000
FDIM = 128

NC = 2
NS = 16
NW = NC * NS
CK = 128
RCH = 16
NRND = 5
NCH = RCH * NRND
E_PAD = NW * CK * NCH
ACC_ROWS = 10240
ZROWS = 64
ROWS_PER_TILE = ACC_ROWS // NS


@functools.partial(
    pl.kernel,
    out_type=jax.ShapeDtypeStruct((NC, ACC_ROWS, FDIM), jnp.float32),
    mesh=plsc.VectorSubcoreMesh(
        core_axis_name="c", subcore_axis_name="s", num_cores=NC, num_subcores=NS
    ),
    scratch_types=[
        pltpu.VMEM((RCH, CK), jnp.int32),
        pltpu.VMEM((RCH, CK), jnp.int32),
        pltpu.VMEM((RCH, CK), jnp.int32),
        pltpu.VMEM((RCH, CK), jnp.int32),
        pltpu.VMEM((CK, FDIM), jnp.float32),
        pltpu.VMEM((CK, FDIM), jnp.float32),
        pltpu.VMEM((ZROWS, FDIM), jnp.float32),
        pltpu.VMEM_SHARED((ACC_ROWS, FDIM), jnp.float32),
        pltpu.SemaphoreType.DMA,
        pltpu.SemaphoreType.DMA,
        pltpu.SemaphoreType.DMA,
    ],
)
def _sc_aggregate(
    table_h, src_h, dst_h, out_h,
    sidx0, sidx1, didx0, didx1, rows_a, rows_b, zbuf_v, acc_s, sem_a, sem_b, sem_i,
):
    c = lax.axis_index("c")
    s = lax.axis_index("s")
    wid = s * NC + c

    zvec = jnp.zeros((16,), jnp.float32)

    def zb_body(k, carry):
        zbuf_v[k // 8, pl.ds((k % 8) * 16, 16)] = zvec
        return carry

    lax.fori_loop(0, ZROWS * 8, zb_body, 0)

    base = s * ROWS_PER_TILE

    def zc_body(k, carry):
        pltpu.sync_copy(zbuf_v, acc_s.at[pl.ds(base + k * ZROWS, ZROWS)])
        return carry

    lax.fori_loop(0, ROWS_PER_TILE // ZROWS, zc_body, 0)
    plsc.subcore_barrier()

    sbufs = (sidx0, sidx1)
    dbufs = (didx0, didx1)

    def i_start(r, sb, db):
        d1 = pltpu.async_copy(src_h.at[wid, pl.ds(r * RCH, RCH)], sb, sem_i)
        d2 = pltpu.async_copy(dst_h.at[wid, pl.ds(r * RCH, RCH)], db, sem_i)
        return (d1, d2)

    pending = i_start(0, sidx0, didx0)

    rows = (rows_a, rows_b)
    sems = (sem_a, sem_b)
    U = 8

    for r in range(NRND):
        sb, db = sbufs[r % 2], dbufs[r % 2]
        for d in pending:
            d.wait()
        if r + 1 < NRND:
            pending = i_start(r + 1, sbufs[(r + 1) % 2], dbufs[(r + 1) % 2])

        def pipe_body(i, carry, sb=sb, db=db):
            g = i * U
            descs = [
                pltpu.async_copy(table_h.at[sb.at[g]], rows[0], sems[0]),
                pltpu.async_copy(table_h.at[sb.at[g + 1]], rows[1], sems[1]),
            ]
            for k in range(U):
                b = k % 2
                descs[b].wait()
                pltpu.sync_copy(rows[b], acc_s.at[db.at[g + k]], add=True)
                if k + 2 < U:
                    descs[b] = pltpu.async_copy(
                        table_h.at[sb.at[g + k + 2]], rows[b], sems[b]
                    )
            return carry

        lax.fori_loop(0, RCH // U, pipe_body, 0)
    plsc.subcore_barrier()

    pltpu.sync_copy(
        acc_s.at[pl.ds(base, ROWS_PER_TILE)],
        out_h.at[c, pl.ds(base, ROWS_PER_TILE)],
    )


def _combine_body(p0_ref, p1_ref, x_ref, wr_ref, wt_ref, b_ref, o_ref):
    agg = p0_ref[0] + p1_ref[0]
    o_ref[...] = (
        jnp.dot(agg, wr_ref[...], preferred_element_type=jnp.float32)
        + jnp.dot(x_ref[...], wt_ref[...], preferred_element_type=jnp.float32)
        + b_ref[...]
    )


def _root_body(x_ref, wt_ref, b_ref, o_ref):
    o_ref[...] = (
        jnp.dot(x_ref[...], wt_ref[...], preferred_element_type=jnp.float32)
        + b_ref[...]
    )


def _rel_body(p0_ref, p1_ref, r_ref, wr_ref, o_ref):
    agg = p0_ref[0] + p1_ref[0]
    o_ref[...] = (
        jnp.dot(agg, wr_ref[...], preferred_element_type=jnp.float32) + r_ref[...]
    )


_BR = 2000


def _combine(parts, x, w_rel, w_root, b):
    return pl.pallas_call(
        _combine_body,
        grid=(N_NODES // _BR,),
        in_specs=[
            pl.BlockSpec((1, _BR, FDIM), lambda i: (0, i, 0)),
            pl.BlockSpec((1, _BR, FDIM), lambda i: (1, i, 0)),
            pl.BlockSpec((_BR, FDIM), lambda i: (i, 0)),
            pl.BlockSpec((FDIM, FDIM), lambda i: (0, 0)),
            pl.BlockSpec((FDIM, FDIM), lambda i: (0, 0)),
            pl.BlockSpec((1, FDIM), lambda i: (0, 0)),
        ],
        out_specs=pl.BlockSpec((_BR, FDIM), lambda i: (i, 0)),
        out_shape=jax.ShapeDtypeStruct((N_NODES, FDIM), jnp.float32),
    )(parts, parts, x, w_rel, w_root, b)


def _root(x, w_root, b):
    return pl.pallas_call(
        _root_body,
        grid=(N_NODES // _BR,),
        in_specs=[
            pl.BlockSpec((_BR, FDIM), lambda i: (i, 0)),
            pl.BlockSpec((FDIM, FDIM), lambda i: (0, 0)),
            pl.BlockSpec((1, FDIM), lambda i: (0, 0)),
        ],
        out_specs=pl.BlockSpec((_BR, FDIM), lambda i: (i, 0)),
        out_shape=jax.ShapeDtypeStruct((N_NODES, FDIM), jnp.float32),
    )(x, w_root, b)


def _rel(parts, root_term, w_rel):
    return pl.pallas_call(
        _rel_body,
        grid=(N_NODES // _BR,),
        in_specs=[
            pl.BlockSpec((1, _BR, FDIM), lambda i: (0, i, 0)),
            pl.BlockSpec((1, _BR, FDIM), lambda i: (1, i, 0)),
            pl.BlockSpec((_BR, FDIM), lambda i: (i, 0)),
            pl.BlockSpec((FDIM, FDIM), lambda i: (0, 0)),
        ],
        out_specs=pl.BlockSpec((_BR, FDIM), lambda i: (i, 0)),
        out_shape=jax.ShapeDtypeStruct((N_NODES, FDIM), jnp.float32),
    )(parts, parts, root_term, w_rel)


def kernel(x, edge_index, W_rel1, b_rel1, W_root1, W_rel2, b_rel2, W_root2):
    pad = E_PAD - E_EDGES
    ar = jnp.arange(pad, dtype=edge_index.dtype)
    src = jnp.concatenate([edge_index[0], ar % N_NODES])
    dst = jnp.concatenate([edge_index[1], N_NODES + ar % (ACC_ROWS - N_NODES)])
    src_r = src.reshape(NCH, CK, NW).transpose(2, 0, 1)
    dst_r = dst.reshape(NCH, CK, NW).transpose(2, 0, 1)

    b1 = b_rel1.reshape(1, FDIM)
    b2 = b_rel2.reshape(1, FDIM)

    parts1 = _sc_aggregate(x, src_r, dst_r)
    root1 = _root(x, W_root1, b1)
    h = _rel(parts1, root1, W_rel1)
    parts2 = _sc_aggregate(h, src_r, dst_r)
    return _combine(parts2, h, W_rel2, W_root2, b2)


# speedup vs baseline: 1.0075x; 1.0075x over previous
import functools

import jax
import jax.numpy as jnp
from jax import lax
from jax.experimental import pallas as pl
from jax.experimental.pallas import tpu as pltpu
from jax.experimental.pallas import tpu_sc as plsc

N_NODES = 10000
E_EDGES = 320000
FDIM = 128

NC = 2
NS = 16
NW = NC * NS
CK = 128
RCH = 16
NRND = 5
NCH = RCH * NRND
E_PAD = NW * CK * NCH
ACC_ROWS = 10240
ZROWS = 64
ROWS_PER_TILE = ACC_ROWS // NS


@functools.partial(
    pl.kernel,
    out_type=jax.ShapeDtypeStruct((NC, ACC_ROWS, FDIM), jnp.float32),
    mesh=plsc.VectorSubcoreMesh(
        core_axis_name="c", subcore_axis_name="s", num_cores=NC, num_subcores=NS
    ),
    scratch_types=[
        pltpu.VMEM((RCH, CK), jnp.int32),
        pltpu.VMEM((RCH, CK), jnp.int32),
        pltpu.VMEM((RCH, CK), jnp.int32),
        pltpu.VMEM((RCH, CK), jnp.int32),
        pltpu.VMEM((CK, FDIM), jnp.float32),
        pltpu.VMEM((CK, FDIM), jnp.float32),
        pltpu.VMEM((ZROWS, FDIM), jnp.float32),
        pltpu.VMEM_SHARED((ACC_ROWS, FDIM), jnp.float32),
        pltpu.SemaphoreType.DMA,
        pltpu.SemaphoreType.DMA,
        pltpu.SemaphoreType.DMA,
    ],
)
def _sc_aggregate(
    table_h, src_h, dst_h, out_h,
    sidx0, sidx1, didx0, didx1, rows_a, rows_b, zbuf_v, acc_s, sem_a, sem_b, sem_i,
):
    c = lax.axis_index("c")
    s = lax.axis_index("s")
    wid = s * NC + c

    sbufs = (sidx0, sidx1)
    dbufs = (didx0, didx1)

    def i_start(r, sb, db):
        d1 = pltpu.async_copy(src_h.at[wid, pl.ds(r * RCH, RCH)], sb, sem_i)
        d2 = pltpu.async_copy(dst_h.at[wid, pl.ds(r * RCH, RCH)], db, sem_i)
        return (d1, d2)

    pending = i_start(0, sidx0, didx0)

    zvec = jnp.zeros((16,), jnp.float32)

    def zb_body(k, carry):
        zbuf_v[k // 8, pl.ds((k % 8) * 16, 16)] = zvec
        return carry

    lax.fori_loop(0, ZROWS * 8, zb_body, 0)

    base = s * ROWS_PER_TILE

    def zc_body(k, carry):
        pltpu.sync_copy(zbuf_v, acc_s.at[pl.ds(base + k * ZROWS, ZROWS)])
        return carry

    lax.fori_loop(0, ROWS_PER_TILE // ZROWS, zc_body, 0)
    plsc.subcore_barrier()

    rows = (rows_a, rows_b)
    sems = (sem_a, sem_b)
    U = 8

    for r in range(NRND):
        sb, db = sbufs[r % 2], dbufs[r % 2]
        for d in pending:
            d.wait()
        if r + 1 < NRND:
            pending = i_start(r + 1, sbufs[(r + 1) % 2], dbufs[(r + 1) % 2])

        def pipe_body(i, carry, sb=sb, db=db):
            g = i * U
            descs = [
                pltpu.async_copy(table_h.at[sb.at[g]], rows[0], sems[0]),
                pltpu.async_copy(table_h.at[sb.at[g + 1]], rows[1], sems[1]),
            ]
            for k in range(U):
                b = k % 2
                descs[b].wait()
                pltpu.sync_copy(rows[b], acc_s.at[db.at[g + k]], add=True)
                if k + 2 < U:
                    descs[b] = pltpu.async_copy(
                        table_h.at[sb.at[g + k + 2]], rows[b], sems[b]
                    )
            return carry

        lax.fori_loop(0, RCH // U, pipe_body, 0)
    plsc.subcore_barrier()

    pltpu.sync_copy(
        acc_s.at[pl.ds(base, ROWS_PER_TILE)],
        out_h.at[c, pl.ds(base, ROWS_PER_TILE)],
    )


def _combine_body(p0_ref, p1_ref, x_ref, wr_ref, wt_ref, b_ref, o_ref):
    agg = p0_ref[0] + p1_ref[0]
    o_ref[...] = (
        jnp.dot(agg, wr_ref[...], preferred_element_type=jnp.float32)
        + jnp.dot(x_ref[...], wt_ref[...], preferred_element_type=jnp.float32)
        + b_ref[...]
    )


def _root_body(x_ref, wt_ref, b_ref, o_ref):
    o_ref[...] = (
        jnp.dot(x_ref[...], wt_ref[...], preferred_element_type=jnp.float32)
        + b_ref[...]
    )


def _rel_body(p0_ref, p1_ref, r_ref, wr_ref, o_ref):
    agg = p0_ref[0] + p1_ref[0]
    o_ref[...] = (
        jnp.dot(agg, wr_ref[...], preferred_element_type=jnp.float32) + r_ref[...]
    )


_BR = 2000


def _combine(parts, x, w_rel, w_root, b):
    return pl.pallas_call(
        _combine_body,
        grid=(N_NODES // _BR,),
        in_specs=[
            pl.BlockSpec((1, _BR, FDIM), lambda i: (0, i, 0)),
            pl.BlockSpec((1, _BR, FDIM), lambda i: (1, i, 0)),
            pl.BlockSpec((_BR, FDIM), lambda i: (i, 0)),
            pl.BlockSpec((FDIM, FDIM), lambda i: (0, 0)),
            pl.BlockSpec((FDIM, FDIM), lambda i: (0, 0)),
            pl.BlockSpec((1, FDIM), lambda i: (0, 0)),
        ],
        out_specs=pl.BlockSpec((_BR, FDIM), lambda i: (i, 0)),
        out_shape=jax.ShapeDtypeStruct((N_NODES, FDIM), jnp.float32),
    )(parts, parts, x, w_rel, w_root, b)


def _root(x, w_root, b):
    return pl.pallas_call(
        _root_body,
        grid=(N_NODES // _BR,),
        in_specs=[
            pl.BlockSpec((_BR, FDIM), lambda i: (i, 0)),
            pl.BlockSpec((FDIM, FDIM), lambda i: (0, 0)),
            pl.BlockSpec((1, FDIM), lambda i: (0, 0)),
        ],
        out_specs=pl.BlockSpec((_BR, FDIM), lambda i: (i, 0)),
        out_shape=jax.ShapeDtypeStruct((N_NODES, FDIM), jnp.float32),
    )(x, w_root, b)


def _rel(parts, root_term, w_rel):
    return pl.pallas_call(
        _rel_body,
        grid=(N_NODES // _BR,),
        in_specs=[
            pl.BlockSpec((1, _BR, FDIM), lambda i: (0, i, 0)),
            pl.BlockSpec((1, _BR, FDIM), lambda i: (1, i, 0)),
            pl.BlockSpec((_BR, FDIM), lambda i: (i, 0)),
            pl.BlockSpec((FDIM, FDIM), lambda i: (0, 0)),
        ],
        out_specs=pl.BlockSpec((_BR, FDIM), lambda i: (i, 0)),
        out_shape=jax.ShapeDtypeStruct((N_NODES, FDIM), jnp.float32),
    )(parts, parts, root_term, w_rel)


def kernel(x, edge_index, W_rel1, b_rel1, W_root1, W_rel2, b_rel2, W_root2):
    pad = E_PAD - E_EDGES
    ar = jnp.arange(pad, dtype=edge_index.dtype)
    src = jnp.concatenate([edge_index[0], ar % N_NODES])
    dst = jnp.concatenate([edge_index[1], N_NODES + ar % (ACC_ROWS - N_NODES)])
    src_r = src.reshape(NCH, CK, NW).transpose(2, 0, 1)
    dst_r = dst.reshape(NCH, CK, NW).transpose(2, 0, 1)

    b1 = b_rel1.reshape(1, FDIM)
    b2 = b_rel2.reshape(1, FDIM)

    parts1 = _sc_aggregate(x, src_r, dst_r)
    h = _combine(parts1, x, W_rel1, W_root1, b1)
    parts2 = _sc_aggregate(h, src_r, dst_r)
    return _combine(parts2, h, W_rel2, W_root2, b2)


# speedup vs baseline: 1.0092x; 1.0017x over previous
import functools

import jax
import jax.numpy as jnp
from jax import lax
from jax.experimental import pallas as pl
from jax.experimental.pallas import tpu as pltpu
from jax.experimental.pallas import tpu_sc as plsc

N_NODES = 10000
E_EDGES = 320000
FDIM = 128

NC = 2
NS = 16
NW = NC * NS
CK = 128
RCH = 16
NRND = 5
NCH = RCH * NRND
E_PAD = NW * CK * NCH
ACC_ROWS = 10240
ZROWS = 64
ROWS_PER_TILE = ACC_ROWS // NS


@functools.partial(
    pl.kernel,
    out_type=jax.ShapeDtypeStruct((NC, ACC_ROWS, FDIM), jnp.float32),
    mesh=plsc.VectorSubcoreMesh(
        core_axis_name="c", subcore_axis_name="s", num_cores=NC, num_subcores=NS
    ),
    scratch_types=[
        pltpu.VMEM((RCH, CK), jnp.int32),
        pltpu.VMEM((RCH, CK), jnp.int32),
        pltpu.VMEM((RCH, CK), jnp.int32),
        pltpu.VMEM((RCH, CK), jnp.int32),
        pltpu.VMEM((CK, FDIM), jnp.float32),
        pltpu.VMEM((CK, FDIM), jnp.float32),
        pltpu.VMEM((ZROWS, FDIM), jnp.float32),
        pltpu.VMEM_SHARED((ACC_ROWS, FDIM), jnp.float32),
        pltpu.SemaphoreType.DMA,
        pltpu.SemaphoreType.DMA,
        pltpu.SemaphoreType.DMA,
    ],
)
def _sc_aggregate(
    table_h, src_h, dst_h, out_h,
    sidx0, sidx1, didx0, didx1, rows_a, rows_b, zbuf_v, acc_s, sem_a, sem_b, sem_i,
):
    c = lax.axis_index("c")
    s = lax.axis_index("s")
    wid = s * NC + c

    sbufs = (sidx0, sidx1)
    dbufs = (didx0, didx1)

    def i_start(r, sb, db):
        d1 = pltpu.async_copy(src_h.at[wid, pl.ds(r * RCH, RCH)], sb, sem_i)
        d2 = pltpu.async_copy(dst_h.at[wid, pl.ds(r * RCH, RCH)], db, sem_i)
        return (d1, d2)

    pending = i_start(0, sidx0, didx0)

    zvec = jnp.zeros((16,), jnp.float32)

    def zb_body(k, carry):
        zbuf_v[k // 8, pl.ds((k % 8) * 16, 16)] = zvec
        return carry

    lax.fori_loop(0, ZROWS * 8, zb_body, 0)

    base = s * ROWS_PER_TILE

    zdescs = [
        pltpu.async_copy(
            zbuf_v, acc_s.at[pl.ds(base + k * ZROWS, ZROWS)], (sem_a, sem_b)[k % 2]
        )
        for k in range(ROWS_PER_TILE // ZROWS)
    ]
    for d in zdescs:
        d.wait()
    plsc.subcore_barrier()

    rows = (rows_a, rows_b)
    sems = (sem_a, sem_b)
    U = 8

    for r in range(NRND):
        sb, db = sbufs[r % 2], dbufs[r % 2]
        for d in pending:
            d.wait()
        if r + 1 < NRND:
            pending = i_start(r + 1, sbufs[(r + 1) % 2], dbufs[(r + 1) % 2])

        def pipe_body(i, carry, sb=sb, db=db):
            g = i * U
            descs = [
                pltpu.async_copy(table_h.at[sb.at[g]], rows[0], sems[0]),
                pltpu.async_copy(table_h.at[sb.at[g + 1]], rows[1], sems[1]),
            ]
            for k in range(U):
                b = k % 2
                descs[b].wait()
                pltpu.sync_copy(rows[b], acc_s.at[db.at[g + k]], add=True)
                if k + 2 < U:
                    descs[b] = pltpu.async_copy(
                        table_h.at[sb.at[g + k + 2]], rows[b], sems[b]
                    )
            return carry

        lax.fori_loop(0, RCH // U, pipe_body, 0)
    plsc.subcore_barrier()

    pltpu.sync_copy(
        acc_s.at[pl.ds(base, ROWS_PER_TILE)],
        out_h.at[c, pl.ds(base, ROWS_PER_TILE)],
    )


def _combine_body(p0_ref, p1_ref, x_ref, wr_ref, wt_ref, b_ref, o_ref):
    agg = p0_ref[0] + p1_ref[0]
    o_ref[...] = (
        jnp.dot(agg, wr_ref[...], preferred_element_type=jnp.float32)
        + jnp.dot(x_ref[...], wt_ref[...], preferred_element_type=jnp.float32)
        + b_ref[...]
    )


_BR = 2000


def _combine(parts, x, w_rel, w_root, b):
    return pl.pallas_call(
        _combine_body,
        grid=(N_NODES // _BR,),
        in_specs=[
            pl.BlockSpec((1, _BR, FDIM), lambda i: (0, i, 0)),
            pl.BlockSpec((1, _BR, FDIM), lambda i: (1, i, 0)),
            pl.BlockSpec((_BR, FDIM), lambda i: (i, 0)),
            pl.BlockSpec((FDIM, FDIM), lambda i: (0, 0)),
            pl.BlockSpec((FDIM, FDIM), lambda i: (0, 0)),
            pl.BlockSpec((1, FDIM), lambda i: (0, 0)),
        ],
        out_specs=pl.BlockSpec((_BR, FDIM), lambda i: (i, 0)),
        out_shape=jax.ShapeDtypeStruct((N_NODES, FDIM), jnp.float32),
    )(parts, parts, x, w_rel, w_root, b)


def kernel(x, edge_index, W_rel1, b_rel1, W_root1, W_rel2, b_rel2, W_root2):
    pad = E_PAD - E_EDGES
    ar = jnp.arange(pad, dtype=edge_index.dtype)
    src = jnp.concatenate([edge_index[0], ar % N_NODES])
    dst = jnp.concatenate([edge_index[1], N_NODES + ar % (ACC_ROWS - N_NODES)])
    src_r = src.reshape(NCH, CK, NW).transpose(2, 0, 1)
    dst_r = dst.reshape(NCH, CK, NW).transpose(2, 0, 1)

    b1 = b_rel1.reshape(1, FDIM)
    b2 = b_rel2.reshape(1, FDIM)

    parts1 = _sc_aggregate(x, src_r, dst_r)
    h = _combine(parts1, x, W_rel1, W_root1, b1)
    parts2 = _sc_aggregate(h, src_r, dst_r)
    return _combine(parts2, h, W_rel2, W_root2, b2)
